# baseline (speedup 1.0000x reference)
import dataclasses
import functools

import jax
import jax.numpy as jnp
from jax import lax
from jax.experimental import pallas as pl
from jax.experimental.pallas import tpu as pltpu
from jax.experimental.pallas import tpu_sc as plsc

_N = 1_000_000
_ROWS = _N // 16
_R_SC = 100
_BLOCKS = _ROWS // _R_SC
_OBS_VAL = 1000.0


def _sc_indicator(x3):
    mesh = plsc.VectorSubcoreMesh(core_axis_name="c", subcore_axis_name="s")
    cp = pltpu.CompilerParams()
    if "needs_layout_passes" in pltpu.CompilerParams.__dataclass_fields__:
        cp = dataclasses.replace(cp, needs_layout_passes=False)

    @functools.partial(
        pl.kernel,
        out_type=jax.ShapeDtypeStruct((_BLOCKS, _R_SC, 16), jnp.float32),
        mesh=mesh,
        compiler_params=cp,
    )
    def sc_kernel(x_hbm, o_hbm):
        def body(x_vmem, o_vmem):
            idx_e = lax.iota(jnp.int32, 16) * 2
            idx_o = idx_e + 1

            @pl.loop(0, _R_SC)
            def _(r):
                row = x_vmem.at[0, r]
                e = plsc.load_gather(row, [idx_e])
                o = plsc.load_gather(row, [idx_o])
                m = (jnp.abs(e) <= 3.0) & (jnp.abs(o) <= 1.5)
                o_vmem[0, r, :] = jnp.where(m, _OBS_VAL, 0.0).astype(jnp.float32)

        pltpu.emit_pipeline(
            body,
            grid=(_BLOCKS,),
            in_specs=[pl.BlockSpec((1, _R_SC, 32), lambda i: (i, 0, 0))],
            out_specs=[pl.BlockSpec((1, _R_SC, 16), lambda i: (i, 0, 0))],
            core_axis_name=("c", "s"),
            dimension_semantics=(pltpu.PARALLEL,),
        )(x_hbm, o_hbm)

    return sc_kernel(x3)


def kernel(x):
    x3 = x.reshape(_BLOCKS, _R_SC, 32)
    out = _sc_indicator(x3)
    return out.reshape(_N, 1)
